# baseline (speedup 1.0000x reference)
import functools

import jax
import jax.numpy as jnp
from jax import lax
from jax.experimental import pallas as pl
from jax.experimental.pallas import tpu as pltpu
from jax.experimental.pallas import tpu_sc as plsc

EMBED_DIM = 32
BATCH = 16384
HIST = 50
NUM_WORKERS = 32
CHUNK = 128
NBLK = BATCH // CHUNK
NUNIT = HIST * NBLK
U_PER_W = NUNIT // NUM_WORKERS

_mesh = plsc.VectorSubcoreMesh(core_axis_name="c", subcore_axis_name="s")


@functools.partial(
    pl.kernel,
    mesh=_mesh,
    compiler_params=pltpu.CompilerParams(
        use_tc_tiling_on_sc=False, needs_layout_passes=False),
    out_type=[
        jax.ShapeDtypeStruct((HIST, EMBED_DIM, BATCH), jnp.float32),
        jax.ShapeDtypeStruct((HIST, EMBED_DIM, BATCH), jnp.float32),
    ],
    scratch_types=[
        pltpu.VMEM((CHUNK,), jnp.int32),
        pltpu.VMEM((CHUNK, EMBED_DIM), jnp.float32),
        pltpu.VMEM((CHUNK, EMBED_DIM), jnp.float32),
        pltpu.VMEM((EMBED_DIM, CHUNK), jnp.float32),
        pltpu.VMEM((EMBED_DIM, CHUNK), jnp.float32),
        pltpu.SemaphoreType.DMA,
    ],
)
def _gather_t_kernel(idx_hbm, loc_hbm, scale_hbm, out_loc, out_scale,
                     idx_v, loc_rows, scale_rows, loc_t, scale_t, sem):
    wid = lax.axis_index("s") * 2 + lax.axis_index("c")

    def unit(t, carry):
        u = wid * U_PER_W + t
        h = u // NBLK
        blk = u % NBLK
        b0 = blk * CHUNK
        pltpu.sync_copy(idx_hbm.at[h, pl.ds(b0, CHUNK)], idx_v)
        cp1 = pltpu.async_copy(loc_hbm.at[idx_v], loc_rows, sem)
        cp2 = pltpu.async_copy(scale_hbm.at[idx_v], scale_rows, sem)
        cp1.wait()
        cp2.wait()
        iota = lax.iota(jnp.int32, 16)
        for g in range(CHUNK // 16):
            rows = iota + (g * 16)
            for d in range(EMBED_DIM):
                cols = jnp.full((16,), d, jnp.int32)
                loc_t[d, pl.ds(g * 16, 16)] = plsc.load_gather(
                    loc_rows, [rows, cols])
                scale_t[d, pl.ds(g * 16, 16)] = plsc.load_gather(
                    scale_rows, [rows, cols])
        pltpu.sync_copy(loc_t, out_loc.at[h, :, pl.ds(b0, CHUNK)])
        pltpu.sync_copy(scale_t, out_scale.at[h, :, pl.ds(b0, CHUNK)])
        return carry

    lax.fori_loop(0, U_PER_W, unit, 0)


def kernel(inputs, loc_table, scale_table):
    idx_t = inputs.astype(jnp.int32).T
    out_loc, out_scale = _gather_t_kernel(idx_t, loc_table, scale_table)
    return (jnp.transpose(out_loc, (2, 0, 1)),
            jnp.transpose(out_scale, (2, 0, 1)))


# speedup vs baseline: 1.7508x; 1.7508x over previous
import functools

import jax
import jax.numpy as jnp
from jax import lax
from jax.experimental import pallas as pl
from jax.experimental.pallas import tpu as pltpu
from jax.experimental.pallas import tpu_sc as plsc

EMBED_DIM = 32
BATCH = 16384
HIST = 50
NUM_WORKERS = 32
CHUNK = 128
B_PER_W = BATCH // NUM_WORKERS
BLK_PER_W = B_PER_W // CHUNK
U_PER_W = HIST * BLK_PER_W
NBUF = 4
PITCH = CHUNK + 1

_mesh = plsc.VectorSubcoreMesh(core_axis_name="c", subcore_axis_name="s")


@functools.partial(
    pl.kernel,
    mesh=_mesh,
    compiler_params=pltpu.CompilerParams(
        use_tc_tiling_on_sc=False, needs_layout_passes=False),
    out_type=[
        jax.ShapeDtypeStruct((HIST, EMBED_DIM, BATCH), jnp.float32),
        jax.ShapeDtypeStruct((HIST, EMBED_DIM, BATCH), jnp.float32),
    ],
    scratch_types=[
        pltpu.VMEM((HIST, B_PER_W), jnp.int32),
        pltpu.VMEM((NBUF, CHUNK, EMBED_DIM), jnp.float32),
        pltpu.VMEM((NBUF, CHUNK, EMBED_DIM), jnp.float32),
        pltpu.VMEM((EMBED_DIM, PITCH), jnp.float32),
        pltpu.VMEM((EMBED_DIM, PITCH), jnp.float32),
        pltpu.SemaphoreType.DMA((NBUF,)),
    ],
)
def _gather_t_kernel(idx_hbm, loc_hbm, scale_hbm, out_loc, out_scale,
                     idx_v, loc_rows, scale_rows, loc_t, scale_t, gsem):
    wid = lax.axis_index("s") * 2 + lax.axis_index("c")
    b_base = wid * B_PER_W
    pltpu.sync_copy(idx_hbm.at[:, pl.ds(b_base, B_PER_W)], idx_v)

    def unit_idx(t):
        h = t // BLK_PER_W
        off = (t % BLK_PER_W) * CHUNK
        return h, off

    def fire(t, slot):
        h, off = unit_idx(t)
        isl = idx_v.at[h, pl.ds(off, CHUNK)]
        pltpu.async_copy(loc_hbm.at[isl], loc_rows.at[slot], gsem.at[slot])
        pltpu.async_copy(scale_hbm.at[isl], scale_rows.at[slot], gsem.at[slot])

    def drain(t, slot):
        h, off = unit_idx(t)
        isl = idx_v.at[h, pl.ds(off, CHUNK)]
        pltpu.make_async_copy(
            loc_hbm.at[isl], loc_rows.at[slot], gsem.at[slot]).wait()
        pltpu.make_async_copy(
            scale_hbm.at[isl], scale_rows.at[slot], gsem.at[slot]).wait()

    for t in range(NBUF):
        fire(t, t)

    iota = lax.iota(jnp.int32, 16)
    rows_lo = iota
    rows_hi = iota + 16

    @pl.loop(0, U_PER_W)
    def unit(t):
        slot = lax.rem(t, NBUF)
        drain(t, slot)
        for bb in range(CHUNK):
            cols = jnp.full((16,), bb, jnp.int32)
            plsc.store_scatter(
                loc_t, [rows_lo, cols],
                loc_rows[slot, bb, pl.ds(0, 16)])
            plsc.store_scatter(
                loc_t, [rows_hi, cols],
                loc_rows[slot, bb, pl.ds(16, 16)])
            plsc.store_scatter(
                scale_t, [rows_lo, cols],
                scale_rows[slot, bb, pl.ds(0, 16)])
            plsc.store_scatter(
                scale_t, [rows_hi, cols],
                scale_rows[slot, bb, pl.ds(16, 16)])

        @pl.when(t + NBUF < U_PER_W)
        def refire():
            fire(t + NBUF, slot)

        h, off = unit_idx(t)
        b0 = b_base + off
        pltpu.sync_copy(loc_t.at[:, pl.ds(0, CHUNK)],
                        out_loc.at[h, :, pl.ds(b0, CHUNK)])
        pltpu.sync_copy(scale_t.at[:, pl.ds(0, CHUNK)],
                        out_scale.at[h, :, pl.ds(b0, CHUNK)])


def kernel(inputs, loc_table, scale_table):
    idx_t = inputs.astype(jnp.int32).T
    out_loc, out_scale = _gather_t_kernel(idx_t, loc_table, scale_table)
    return (jnp.transpose(out_loc, (2, 0, 1)),
            jnp.transpose(out_scale, (2, 0, 1)))


# speedup vs baseline: 2.1667x; 1.2375x over previous
import functools

import jax
import jax.numpy as jnp
from jax import lax
from jax.experimental import pallas as pl
from jax.experimental.pallas import tpu as pltpu
from jax.experimental.pallas import tpu_sc as plsc

EMBED_DIM = 32
BATCH = 16384
HIST = 50
NUM_WORKERS = 32
CHUNK = 128
B_PER_W = BATCH // NUM_WORKERS
BLK_PER_W = B_PER_W // CHUNK
U_PER_W = HIST * BLK_PER_W
NBUF = 4
PITCH = CHUNK + 1

_mesh = plsc.VectorSubcoreMesh(core_axis_name="c", subcore_axis_name="s")


@functools.partial(
    pl.kernel,
    mesh=_mesh,
    compiler_params=pltpu.CompilerParams(
        use_tc_tiling_on_sc=False, needs_layout_passes=False),
    out_type=jax.ShapeDtypeStruct((HIST, EMBED_DIM, BATCH), jnp.float32),
    scratch_types=[
        pltpu.VMEM((HIST, B_PER_W), jnp.int32),
        pltpu.VMEM((NBUF, CHUNK, EMBED_DIM), jnp.float32),
        pltpu.VMEM((EMBED_DIM, PITCH), jnp.float32),
        pltpu.SemaphoreType.DMA((NBUF,)),
    ],
)
def _gather_one(idx_hbm, table_hbm, out_hbm, idx_v, rows_v, trans_v, gsem):
    wid = lax.axis_index("s") * 2 + lax.axis_index("c")
    b_base = wid * B_PER_W
    pltpu.sync_copy(idx_hbm.at[:, pl.ds(b_base, B_PER_W)], idx_v)

    def unit_idx(t):
        h = t // BLK_PER_W
        off = (t % BLK_PER_W) * CHUNK
        return h, off

    def fire(t, slot):
        h, off = unit_idx(t)
        isl = idx_v.at[h, pl.ds(off, CHUNK)]
        pltpu.async_copy(table_hbm.at[isl], rows_v.at[slot], gsem.at[slot])

    def drain(t, slot):
        h, off = unit_idx(t)
        isl = idx_v.at[h, pl.ds(off, CHUNK)]
        pltpu.make_async_copy(
            table_hbm.at[isl], rows_v.at[slot], gsem.at[slot]).wait()

    for t in range(NBUF):
        fire(t, t)

    iota = lax.iota(jnp.int32, 16)
    rows_lo = iota
    rows_hi = iota + 16

    @pl.loop(0, U_PER_W)
    def unit(t):
        slot = lax.rem(t, NBUF)
        drain(t, slot)
        for bb in range(CHUNK):
            cols = jnp.full((16,), bb, jnp.int32)
            plsc.store_scatter(
                trans_v, [rows_lo, cols], rows_v[slot, bb, pl.ds(0, 16)])
            plsc.store_scatter(
                trans_v, [rows_hi, cols], rows_v[slot, bb, pl.ds(16, 16)])

        @pl.when(t + NBUF < U_PER_W)
        def refire():
            fire(t + NBUF, slot)

        h, off = unit_idx(t)
        b0 = b_base + off
        pltpu.sync_copy(trans_v.at[:, pl.ds(0, CHUNK)],
                        out_hbm.at[h, :, pl.ds(b0, CHUNK)])


def kernel(inputs, loc_table, scale_table):
    idx_t = inputs.astype(jnp.int32).T
    out_loc = _gather_one(idx_t, loc_table)
    out_scale = _gather_one(idx_t, scale_table)
    return (jnp.transpose(out_loc, (2, 0, 1)),
            jnp.transpose(out_scale, (2, 0, 1)))
